# baseline (speedup 1.0000x reference)
import functools

import jax
import jax.numpy as jnp
from jax import lax
from jax.experimental import pallas as pl
from jax.experimental.pallas import tpu as pltpu
from jax.experimental.pallas import tpu_sc as plsc

N = 10000
NP = 10016
D = 128
E = 320000
LANE = 128
NSC = 2
NSUB = 16
UNITS = 2560
EPAD = UNITS * LANE
SLAB = 8
SPLIT = 256
UPT_B = UNITS // NSUB
UPT_U0 = SPLIT // NSUB
UPT_U1 = (UNITS - SPLIT) // NSUB
DUMMY = NP
ACC_ROWS = NP + 16
RPT = 624
TAIL = N - NSUB * RPT
BLK = 2504


def _tc_body(xu, xi, wmb, wmbb, wmf, wrb, wru, tb, tbb, tf, ii, iu):
    u = xu[...]
    i = xi[...]
    tb[...] = jnp.dot(u, wmb[...], preferred_element_type=jnp.float32)
    tbb[...] = jnp.dot(i, wmbb[...], preferred_element_type=jnp.float32)
    tf[...] = jnp.dot(u, wmf[...], preferred_element_type=jnp.float32)
    ii[...] = jnp.dot(i, wrb[...], preferred_element_type=jnp.float32)
    iu[...] = jnp.dot(u, wru[...], preferred_element_type=jnp.float32)


def _tc_transform(x_user, x_item, wmb, wmbb, wmf, wrb, wru):
    nblk = NP // BLK
    xspec = pl.BlockSpec((BLK, D), lambda i: (i, 0))
    wspec = pl.BlockSpec((D, D), lambda i: (0, 0))
    ospec = pl.BlockSpec((BLK, D), lambda i: (i, 0))
    oshape = jax.ShapeDtypeStruct((NP, D), jnp.float32)
    pad = ((0, NP - N), (0, 0))
    return pl.pallas_call(
        _tc_body,
        grid=(nblk,),
        in_specs=[xspec, xspec, wspec, wspec, wspec, wspec, wspec],
        out_specs=[ospec] * 5,
        out_shape=[oshape] * 5,
    )(jnp.pad(x_user, pad), jnp.pad(x_item, pad), wmb, wmbb, wmf, wrb, wru)


def _add_body(a, b, o):
    o[...] = a[...] + b[...]


def _tc_add(a, b):
    spec = pl.BlockSpec((2000, D), lambda i: (i, 0))
    return pl.pallas_call(
        _add_body,
        grid=(N // 2000,),
        in_specs=[spec, spec],
        out_specs=spec,
        out_shape=jax.ShapeDtypeStruct((N, D), jnp.float32),
    )(a, b)


def _sc_body(tb, tbb, tf, ii, iu, zz,
             sb_b, db_b, sb_bb, db_bb, sb_f, db_f,
             out_i, pu0, pu1,
             src_v, dst_v, rows_a, rows_b, acc, sem_a, sem_b):
    c = lax.axis_index("c")
    s = lax.axis_index("s")
    row0 = s * RPT
    t0 = NSUB * RPT

    def init_acc(init_hbm):
        pltpu.sync_copy(init_hbm.at[pl.ds(row0, RPT)], acc.at[pl.ds(row0, RPT)])
        @pl.when(s == NSUB - 1)
        def _():
            pltpu.sync_copy(init_hbm.at[pl.ds(t0, TAIL)], acc.at[pl.ds(t0, TAIL)])
        plsc.subcore_barrier()

    def copy_out(out_hbm):
        plsc.subcore_barrier()
        pltpu.sync_copy(acc.at[pl.ds(row0, RPT)], out_hbm.at[pl.ds(row0, RPT)])
        @pl.when(s == NSUB - 1)
        def _():
            pltpu.sync_copy(acc.at[pl.ds(t0, TAIL)], out_hbm.at[pl.ds(t0, TAIL)])
        plsc.subcore_barrier()

    def run_edges(tbl, srcb, dstb, ubase, nunits):
        def slab_body(t, carry):
            u0 = ubase + t * SLAB
            pltpu.sync_copy(srcb.at[pl.ds(u0, SLAB)], src_v)
            pltpu.sync_copy(dstb.at[pl.ds(u0, SLAB)], dst_v)
            pltpu.async_copy(tbl.at[src_v.at[0]], rows_a, sem_a)
            for p in range(SLAB // 2):
                j = 2 * p
                pltpu.make_async_copy(tbl.at[src_v.at[j]], rows_a, sem_a).wait()
                pltpu.async_copy(tbl.at[src_v.at[j + 1]], rows_b, sem_b)
                pltpu.sync_copy(rows_a, acc.at[dst_v.at[j]], add=True)
                pltpu.make_async_copy(tbl.at[src_v.at[j + 1]], rows_b,
                                      sem_b).wait()
                if j + 2 < SLAB:
                    pltpu.async_copy(tbl.at[src_v.at[j + 2]], rows_a, sem_a)
                pltpu.sync_copy(rows_b, acc.at[dst_v.at[j + 1]], add=True)
            return carry

        lax.fori_loop(0, nunits // SLAB, slab_body, 0, unroll=False)

    @pl.when(c == 0)
    def _():
        init_acc(ii)
        run_edges(tb, sb_b, db_b, s * UPT_B, UPT_B)
        copy_out(out_i)
        init_acc(zz)
        run_edges(tbb, sb_bb, db_bb, s * UPT_U0, UPT_U0)
        run_edges(tf, sb_f, db_f, s * UPT_U0, UPT_U0)
        copy_out(pu0)

    @pl.when(c == 1)
    def _():
        init_acc(iu)
        run_edges(tbb, sb_bb, db_bb, SPLIT + s * UPT_U1, UPT_U1)
        run_edges(tf, sb_f, db_f, SPLIT + s * UPT_U1, UPT_U1)
        copy_out(pu1)


_sc_aggregate = functools.partial(
    pl.kernel,
    out_type=(
        jax.ShapeDtypeStruct((N, D), jnp.float32),
        jax.ShapeDtypeStruct((N, D), jnp.float32),
        jax.ShapeDtypeStruct((N, D), jnp.float32),
    ),
    mesh=plsc.VectorSubcoreMesh(core_axis_name="c", subcore_axis_name="s"),
    scratch_types=[
        pltpu.VMEM((SLAB, LANE), jnp.int32),
        pltpu.VMEM((SLAB, LANE), jnp.int32),
        pltpu.VMEM((LANE, D), jnp.float32),
        pltpu.VMEM((LANE, D), jnp.float32),
        pltpu.VMEM_SHARED((ACC_ROWS, D), jnp.float32),
        pltpu.SemaphoreType.DMA,
        pltpu.SemaphoreType.DMA,
    ],
)(_sc_body)


def _prep_edges(ei):
    src = ei[0].astype(jnp.int32)
    dst = ei[1].astype(jnp.int32)
    pad = EPAD - E
    src_p = jnp.concatenate([src, jnp.zeros((pad,), jnp.int32)])
    dst_p = jnp.concatenate([dst, jnp.full((pad,), DUMMY, jnp.int32)])
    return src_p.reshape(UNITS, LANE), dst_p.reshape(UNITS, LANE)


def kernel(x_user, x_item, edge_index_buys, edge_index_bought_by,
           edge_index_follows, W_msg_buys, W_root_buys, W_msg_bought_by,
           W_root_bought_by, W_msg_follows, W_root_follows):
    wru = W_root_bought_by + W_root_follows
    tb, tbb, tf, ii, iu = _tc_transform(
        x_user, x_item, W_msg_buys, W_msg_bought_by, W_msg_follows,
        W_root_buys, wru)
    sb_b, db_b = _prep_edges(edge_index_buys)
    sb_bb, db_bb = _prep_edges(edge_index_bought_by)
    sb_f, db_f = _prep_edges(edge_index_follows)
    zz = jnp.zeros((NP, D), jnp.float32)
    out_item, pu0, pu1 = _sc_aggregate(tb, tbb, tf, ii, iu, zz,
                                       sb_b, db_b, sb_bb, db_bb, sb_f, db_f)
    out_user = _tc_add(pu0, pu1)
    return (out_user, out_item)


# speedup vs baseline: 1.0595x; 1.0595x over previous
import functools

import jax
import jax.numpy as jnp
from jax import lax
from jax.experimental import pallas as pl
from jax.experimental.pallas import tpu as pltpu
from jax.experimental.pallas import tpu_sc as plsc

N = 10000
NP = 10016
D = 128
E = 320000
LANE = 128
NSC = 2
NSUB = 16
UNITS = 2560
EPAD = UNITS * LANE
SLAB = 16
SPLIT = 512
UPT_B = UNITS // NSUB
UPT_U0 = SPLIT // NSUB
UPT_U1 = (UNITS - SPLIT) // NSUB
DUMMY = NP
ACC_ROWS = NP + 16
RPT = 624
TAIL = N - NSUB * RPT
BLK = 2504


def _tc_body(xu, xi, wmb, wmbb, wmf, wrb, wru, tb, tbb, tf, ii, iu):
    u = xu[...]
    i = xi[...]
    tb[...] = jnp.dot(u, wmb[...], preferred_element_type=jnp.float32)
    tbb[...] = jnp.dot(i, wmbb[...], preferred_element_type=jnp.float32)
    tf[...] = jnp.dot(u, wmf[...], preferred_element_type=jnp.float32)
    ii[...] = jnp.dot(i, wrb[...], preferred_element_type=jnp.float32)
    iu[...] = jnp.dot(u, wru[...], preferred_element_type=jnp.float32)


def _tc_transform(x_user, x_item, wmb, wmbb, wmf, wrb, wru):
    nblk = NP // BLK
    xspec = pl.BlockSpec((BLK, D), lambda i: (i, 0))
    wspec = pl.BlockSpec((D, D), lambda i: (0, 0))
    ospec = pl.BlockSpec((BLK, D), lambda i: (i, 0))
    oshape = jax.ShapeDtypeStruct((NP, D), jnp.float32)
    pad = ((0, NP - N), (0, 0))
    return pl.pallas_call(
        _tc_body,
        grid=(nblk,),
        in_specs=[xspec, xspec, wspec, wspec, wspec, wspec, wspec],
        out_specs=[ospec] * 5,
        out_shape=[oshape] * 5,
    )(jnp.pad(x_user, pad), jnp.pad(x_item, pad), wmb, wmbb, wmf, wrb, wru)


def _add_body(a, b, o):
    o[...] = a[...] + b[...]


def _tc_add(a, b):
    spec = pl.BlockSpec((2000, D), lambda i: (i, 0))
    return pl.pallas_call(
        _add_body,
        grid=(N // 2000,),
        in_specs=[spec, spec],
        out_specs=spec,
        out_shape=jax.ShapeDtypeStruct((N, D), jnp.float32),
    )(a, b)


def _sc_body(tb, tbb, tf, ii, iu, zz,
             sb_b, db_b, sb_bb, db_bb, sb_f, db_f,
             out_i, pu0, pu1,
             src_v, dst_v, rows_a, rows_b, acc, sem_a, sem_b):
    c = lax.axis_index("c")
    s = lax.axis_index("s")
    row0 = s * RPT
    t0 = NSUB * RPT

    def init_acc(init_hbm):
        pltpu.sync_copy(init_hbm.at[pl.ds(row0, RPT)], acc.at[pl.ds(row0, RPT)])
        @pl.when(s == NSUB - 1)
        def _():
            pltpu.sync_copy(init_hbm.at[pl.ds(t0, TAIL)], acc.at[pl.ds(t0, TAIL)])
        plsc.subcore_barrier()

    def copy_out(out_hbm):
        plsc.subcore_barrier()
        pltpu.sync_copy(acc.at[pl.ds(row0, RPT)], out_hbm.at[pl.ds(row0, RPT)])
        @pl.when(s == NSUB - 1)
        def _():
            pltpu.sync_copy(acc.at[pl.ds(t0, TAIL)], out_hbm.at[pl.ds(t0, TAIL)])
        plsc.subcore_barrier()

    def run_edges(tbl, srcb, dstb, ubase, nunits):
        def slab_body(t, carry):
            u0 = ubase + t * SLAB
            pltpu.sync_copy(srcb.at[pl.ds(u0, SLAB)], src_v)
            pltpu.sync_copy(dstb.at[pl.ds(u0, SLAB)], dst_v)
            pltpu.async_copy(tbl.at[src_v.at[0]], rows_a, sem_a)
            for p in range(SLAB // 2):
                j = 2 * p
                pltpu.make_async_copy(tbl.at[src_v.at[j]], rows_a, sem_a).wait()
                pltpu.async_copy(tbl.at[src_v.at[j + 1]], rows_b, sem_b)
                pltpu.sync_copy(rows_a, acc.at[dst_v.at[j]], add=True)
                pltpu.make_async_copy(tbl.at[src_v.at[j + 1]], rows_b,
                                      sem_b).wait()
                if j + 2 < SLAB:
                    pltpu.async_copy(tbl.at[src_v.at[j + 2]], rows_a, sem_a)
                pltpu.sync_copy(rows_b, acc.at[dst_v.at[j + 1]], add=True)
            return carry

        lax.fori_loop(0, nunits // SLAB, slab_body, 0, unroll=False)

    @pl.when(c == 0)
    def _():
        init_acc(ii)
        run_edges(tb, sb_b, db_b, s * UPT_B, UPT_B)
        copy_out(out_i)
        init_acc(zz)
        run_edges(tbb, sb_bb, db_bb, s * UPT_U0, UPT_U0)
        run_edges(tf, sb_f, db_f, s * UPT_U0, UPT_U0)
        copy_out(pu0)

    @pl.when(c == 1)
    def _():
        init_acc(iu)
        run_edges(tbb, sb_bb, db_bb, SPLIT + s * UPT_U1, UPT_U1)
        run_edges(tf, sb_f, db_f, SPLIT + s * UPT_U1, UPT_U1)
        copy_out(pu1)


_sc_aggregate = functools.partial(
    pl.kernel,
    out_type=(
        jax.ShapeDtypeStruct((N, D), jnp.float32),
        jax.ShapeDtypeStruct((N, D), jnp.float32),
        jax.ShapeDtypeStruct((N, D), jnp.float32),
    ),
    mesh=plsc.VectorSubcoreMesh(core_axis_name="c", subcore_axis_name="s"),
    scratch_types=[
        pltpu.VMEM((SLAB, LANE), jnp.int32),
        pltpu.VMEM((SLAB, LANE), jnp.int32),
        pltpu.VMEM((LANE, D), jnp.float32),
        pltpu.VMEM((LANE, D), jnp.float32),
        pltpu.VMEM_SHARED((ACC_ROWS, D), jnp.float32),
        pltpu.SemaphoreType.DMA,
        pltpu.SemaphoreType.DMA,
    ],
)(_sc_body)


def _prep_edges(ei):
    src = ei[0].astype(jnp.int32)
    dst = ei[1].astype(jnp.int32)
    pad = EPAD - E
    src_p = jnp.concatenate([src, jnp.zeros((pad,), jnp.int32)])
    dst_p = jnp.concatenate([dst, jnp.full((pad,), DUMMY, jnp.int32)])
    return src_p.reshape(UNITS, LANE), dst_p.reshape(UNITS, LANE)


def kernel(x_user, x_item, edge_index_buys, edge_index_bought_by,
           edge_index_follows, W_msg_buys, W_root_buys, W_msg_bought_by,
           W_root_bought_by, W_msg_follows, W_root_follows):
    wru = W_root_bought_by + W_root_follows
    tb, tbb, tf, ii, iu = _tc_transform(
        x_user, x_item, W_msg_buys, W_msg_bought_by, W_msg_follows,
        W_root_buys, wru)
    sb_b, db_b = _prep_edges(edge_index_buys)
    sb_bb, db_bb = _prep_edges(edge_index_bought_by)
    sb_f, db_f = _prep_edges(edge_index_follows)
    zz = jnp.zeros((NP, D), jnp.float32)
    out_item, pu0, pu1 = _sc_aggregate(tb, tbb, tf, ii, iu, zz,
                                       sb_b, db_b, sb_bb, db_bb, sb_f, db_f)
    out_user = _tc_add(pu0, pu1)
    return (out_user, out_item)


# speedup vs baseline: 1.0653x; 1.0054x over previous
import functools

import jax
import jax.numpy as jnp
from jax import lax
from jax.experimental import pallas as pl
from jax.experimental.pallas import tpu as pltpu
from jax.experimental.pallas import tpu_sc as plsc

N = 10000
NP = 10016
D = 128
E = 320000
LANE = 128
NSC = 2
NSUB = 16
UNITS = 2560
EPAD = UNITS * LANE
SLAB = 16
SPLIT = 512
UPT_B = UNITS // NSUB
UPT_U0 = SPLIT // NSUB
UPT_U1 = (UNITS - SPLIT) // NSUB
DUMMY = NP
ACC_ROWS = NP + 16
RPT = 624
TAIL = N - NSUB * RPT
BLK = 2504


def _tc_body(xu, xi, wmb, wmbb, wmf, wrb, wru, tb, tbb, tf, ii, iu):
    u = xu[...]
    i = xi[...]
    tb[...] = jnp.dot(u, wmb[...], preferred_element_type=jnp.float32)
    tbb[...] = jnp.dot(i, wmbb[...], preferred_element_type=jnp.float32)
    tf[...] = jnp.dot(u, wmf[...], preferred_element_type=jnp.float32)
    ii[...] = jnp.dot(i, wrb[...], preferred_element_type=jnp.float32)
    iu[...] = jnp.dot(u, wru[...], preferred_element_type=jnp.float32)


def _tc_transform(x_user, x_item, wmb, wmbb, wmf, wrb, wru):
    nblk = NP // BLK
    xspec = pl.BlockSpec((BLK, D), lambda i: (i, 0))
    wspec = pl.BlockSpec((D, D), lambda i: (0, 0))
    ospec = pl.BlockSpec((BLK, D), lambda i: (i, 0))
    oshape = jax.ShapeDtypeStruct((NP, D), jnp.float32)
    pad = ((0, NP - N), (0, 0))
    return pl.pallas_call(
        _tc_body,
        grid=(nblk,),
        in_specs=[xspec, xspec, wspec, wspec, wspec, wspec, wspec],
        out_specs=[ospec] * 5,
        out_shape=[oshape] * 5,
    )(jnp.pad(x_user, pad), jnp.pad(x_item, pad), wmb, wmbb, wmf, wrb, wru)


def _add_body(a, b, o):
    o[...] = a[...] + b[...]


def _tc_add(a, b):
    spec = pl.BlockSpec((2000, D), lambda i: (i, 0))
    return pl.pallas_call(
        _add_body,
        grid=(N // 2000,),
        in_specs=[spec, spec],
        out_specs=spec,
        out_shape=jax.ShapeDtypeStruct((N, D), jnp.float32),
    )(a, b)


def _sc_body(tb, tbb, tf, ii, iu, zz,
             sb_b, db_b, sb_bb, db_bb, sb_f, db_f,
             out_i, pu0, pu1,
             src_v, dst_v, rows_a, rows_b, acc, sem_a, sem_b):
    c = lax.axis_index("c")
    s = lax.axis_index("s")
    row0 = s * RPT
    t0 = NSUB * RPT

    def init_acc(init_hbm):
        pltpu.sync_copy(init_hbm.at[pl.ds(row0, RPT)], acc.at[pl.ds(row0, RPT)])
        @pl.when(s == NSUB - 1)
        def _():
            pltpu.sync_copy(init_hbm.at[pl.ds(t0, TAIL)], acc.at[pl.ds(t0, TAIL)])
        plsc.subcore_barrier()

    def copy_out(out_hbm):
        plsc.subcore_barrier()
        pltpu.sync_copy(acc.at[pl.ds(row0, RPT)], out_hbm.at[pl.ds(row0, RPT)])
        @pl.when(s == NSUB - 1)
        def _():
            pltpu.sync_copy(acc.at[pl.ds(t0, TAIL)], out_hbm.at[pl.ds(t0, TAIL)])
        plsc.subcore_barrier()

    def run_edges(tbl, srcb, dstb, ubase, nunits):
        def slab_body(t, carry):
            u0 = ubase + t * SLAB
            pltpu.sync_copy(srcb.at[pl.ds(u0, SLAB)], src_v)
            pltpu.sync_copy(dstb.at[pl.ds(u0, SLAB)], dst_v)
            pltpu.async_copy(tbl.at[src_v.at[0]], rows_a, sem_a)
            for p in range(SLAB // 2):
                j = 2 * p
                pltpu.make_async_copy(tbl.at[src_v.at[j]], rows_a, sem_a).wait()
                pltpu.async_copy(tbl.at[src_v.at[j + 1]], rows_b, sem_b)
                pltpu.sync_copy(rows_a, acc.at[dst_v.at[j]], add=True)
                pltpu.make_async_copy(tbl.at[src_v.at[j + 1]], rows_b,
                                      sem_b).wait()
                if j + 2 < SLAB:
                    pltpu.async_copy(tbl.at[src_v.at[j + 2]], rows_a, sem_a)
                pltpu.sync_copy(rows_b, acc.at[dst_v.at[j + 1]], add=True)
            return carry

        lax.fori_loop(0, nunits // SLAB, slab_body, 0, unroll=False)

    @pl.when(c == 1)
    def _():
        init_acc(ii)
        run_edges(tb, sb_b, db_b, s * UPT_B, UPT_B)
        copy_out(out_i)
        init_acc(zz)
        run_edges(tbb, sb_bb, db_bb, s * UPT_U0, UPT_U0)
        run_edges(tf, sb_f, db_f, s * UPT_U0, UPT_U0)
        copy_out(pu0)

    @pl.when(c == 0)
    def _():
        init_acc(iu)
        run_edges(tbb, sb_bb, db_bb, SPLIT + s * UPT_U1, UPT_U1)
        run_edges(tf, sb_f, db_f, SPLIT + s * UPT_U1, UPT_U1)
        copy_out(pu1)


_sc_aggregate = functools.partial(
    pl.kernel,
    out_type=(
        jax.ShapeDtypeStruct((N, D), jnp.float32),
        jax.ShapeDtypeStruct((N, D), jnp.float32),
        jax.ShapeDtypeStruct((N, D), jnp.float32),
    ),
    mesh=plsc.VectorSubcoreMesh(core_axis_name="c", subcore_axis_name="s"),
    scratch_types=[
        pltpu.VMEM((SLAB, LANE), jnp.int32),
        pltpu.VMEM((SLAB, LANE), jnp.int32),
        pltpu.VMEM((LANE, D), jnp.float32),
        pltpu.VMEM((LANE, D), jnp.float32),
        pltpu.VMEM_SHARED((ACC_ROWS, D), jnp.float32),
        pltpu.SemaphoreType.DMA,
        pltpu.SemaphoreType.DMA,
    ],
)(_sc_body)


def _prep_edges(ei):
    src = ei[0].astype(jnp.int32)
    dst = ei[1].astype(jnp.int32)
    pad = EPAD - E
    src_p = jnp.concatenate([src, jnp.zeros((pad,), jnp.int32)])
    dst_p = jnp.concatenate([dst, jnp.full((pad,), DUMMY, jnp.int32)])
    return src_p.reshape(UNITS, LANE), dst_p.reshape(UNITS, LANE)


def kernel(x_user, x_item, edge_index_buys, edge_index_bought_by,
           edge_index_follows, W_msg_buys, W_root_buys, W_msg_bought_by,
           W_root_bought_by, W_msg_follows, W_root_follows):
    wru = W_root_bought_by + W_root_follows
    tb, tbb, tf, ii, iu = _tc_transform(
        x_user, x_item, W_msg_buys, W_msg_bought_by, W_msg_follows,
        W_root_buys, wru)
    sb_b, db_b = _prep_edges(edge_index_buys)
    sb_bb, db_bb = _prep_edges(edge_index_bought_by)
    sb_f, db_f = _prep_edges(edge_index_follows)
    zz = jnp.zeros((NP, D), jnp.float32)
    out_item, pu0, pu1 = _sc_aggregate(tb, tbb, tf, ii, iu, zz,
                                       sb_b, db_b, sb_bb, db_bb, sb_f, db_f)
    out_user = _tc_add(pu0, pu1)
    return (out_user, out_item)
